# baseline (speedup 1.0000x reference)
import functools
import math

import jax
import jax.numpy as jnp
from jax.experimental import pallas as pl

_NEG = -1e9
_STEPS = 4
_TILE = 12800


def _rf_kernel(x_ref, w_ref, o_ref):
    o_ref[...] = jnp.dot(x_ref[...], w_ref[...], preferred_element_type=jnp.float32)


def _u_kernel(node_ref, wt_ref, wh_ref, h_ref, o_ref):
    dn = (((1,), (1,)), ((), ()))
    v_t = jax.lax.dot_general(wt_ref[...], h_ref[...], dn, preferred_element_type=jnp.float32)
    v_h = jax.lax.dot_general(wh_ref[...], h_ref[...], dn, preferred_element_type=jnp.float32)
    v = jnp.concatenate([v_t, v_h], axis=1)
    o_ref[...] = jnp.dot(node_ref[...], v, preferred_element_type=jnp.float32)


def _step_kernel(ptr_ref, rf_ref, g_ref, h_ref, m_ref, s_ref, i_ref, *, tile, n_b, n_e, inv_sqrt_d):
    t = pl.program_id(0)

    @pl.when(t == 0)
    def _():
        m_ref[...] = jnp.full_like(m_ref[...], _NEG)
        s_ref[...] = jnp.zeros_like(s_ref[...])
        i_ref[...] = jnp.full_like(i_ref[...], n_e)

    rows = t * tile + jax.lax.broadcasted_iota(jnp.int32, (tile, 1), 0)
    starts = ptr_ref[0:1, 0:n_b]
    ends = ptr_ref[0:1, 1:n_b + 1]
    mask = (rows >= starts) & (rows < ends)

    scores = (jnp.dot(rf_ref[...], h_ref[...].T, preferred_element_type=jnp.float32)
              + g_ref[...]) * inv_sqrt_d
    s = jnp.where(mask, scores, _NEG)
    tile_m = jnp.max(s, axis=0, keepdims=True)
    tile_s = jnp.sum(jnp.where(mask, jnp.exp(s - tile_m), 0.0), axis=0, keepdims=True)
    cand = jnp.where((s >= tile_m) & mask, jnp.broadcast_to(rows, s.shape), n_e)
    tile_i = jnp.min(cand, axis=0, keepdims=True)

    m_old = m_ref[...]
    s_old = s_ref[...]
    i_old = i_ref[...]
    m_new = jnp.maximum(m_old, tile_m)
    s_ref[...] = s_old * jnp.exp(m_old - m_new) + tile_s * jnp.exp(tile_m - m_new)
    i_ref[...] = jnp.where(tile_m > m_old, tile_i, i_old)
    m_ref[...] = m_new


def kernel(node_tokens, relation_tokens, question_tokens, edge_index, edge_batch, edge_ptr, node_ptr, curr_nodes, W_init, b_init, W_rel, W_tail, W_head, W_stop, b_stop, W_upd, b_upd):
    del node_ptr
    n_nodes, d = node_tokens.shape
    n_e = relation_tokens.shape[0]
    n_b = question_tokens.shape[0]
    inv_sqrt_d = 1.0 / math.sqrt(float(d))
    tile = _TILE
    n_tiles = n_e // tile

    heads = edge_index[0].astype(jnp.int32)
    tails = edge_index[1].astype(jnp.int32)
    edge_batch = edge_batch.astype(jnp.int32)

    rf = pl.pallas_call(
        _rf_kernel,
        grid=(n_tiles,),
        in_specs=[
            pl.BlockSpec((tile, d), lambda i: (i, 0)),
            pl.BlockSpec((d, d), lambda i: (0, 0)),
        ],
        out_specs=pl.BlockSpec((tile, d), lambda i: (i, 0)),
        out_shape=jax.ShapeDtypeStruct((n_e, d), jnp.float32),
    )(relation_tokens, W_rel)

    ptr_pad = jnp.zeros((1, 32), jnp.int32).at[0, :n_b + 1].set(edge_ptr.astype(jnp.int32))

    start_tok = jnp.take(node_tokens, jnp.clip(curr_nodes, 0, None), axis=0)
    start_tok = jnp.where((curr_nodes >= 0)[:, None], start_tok, 0.0)
    hidden = jnp.tanh(jnp.concatenate([question_tokens, start_tok], axis=-1) @ W_init + b_init)
    done = jnp.zeros((n_b,), dtype=bool)

    u_call = pl.pallas_call(
        _u_kernel,
        in_specs=[pl.BlockSpec(node_tokens.shape, lambda: (0, 0)),
                  pl.BlockSpec((d, d), lambda: (0, 0)),
                  pl.BlockSpec((d, d), lambda: (0, 0)),
                  pl.BlockSpec((n_b, d), lambda: (0, 0))],
        out_specs=pl.BlockSpec((n_nodes, 2 * n_b), lambda: (0, 0)),
        out_shape=jax.ShapeDtypeStruct((n_nodes, 2 * n_b), jnp.float32),
    )

    step_call = pl.pallas_call(
        functools.partial(_step_kernel, tile=tile, n_b=n_b, n_e=n_e, inv_sqrt_d=inv_sqrt_d),
        grid=(n_tiles,),
        in_specs=[
            pl.BlockSpec((1, 32), lambda i: (0, 0)),
            pl.BlockSpec((tile, d), lambda i: (i, 0)),
            pl.BlockSpec((tile, 1), lambda i: (i, 0)),
            pl.BlockSpec((n_b, d), lambda i: (0, 0)),
        ],
        out_specs=[
            pl.BlockSpec((1, n_b), lambda i: (0, 0)),
            pl.BlockSpec((1, n_b), lambda i: (0, 0)),
            pl.BlockSpec((1, n_b), lambda i: (0, 0)),
        ],
        out_shape=[
            jax.ShapeDtypeStruct((1, n_b), jnp.float32),
            jax.ShapeDtypeStruct((1, n_b), jnp.float32),
            jax.ShapeDtypeStruct((1, n_b), jnp.int32),
        ],
    )

    flat_all = jnp.concatenate([tails * (2 * n_b) + edge_batch,
                                heads * (2 * n_b) + n_b + edge_batch])

    log_pf_steps = []
    actions_steps = []
    for _ in range(_STEPS):
        u = u_call(node_tokens, W_tail, W_head, hidden)
        g2 = jnp.take(u.reshape(-1), flat_all)
        g = g2[:n_e] + g2[n_e:]
        m2, s2, i2 = step_call(ptr_pad, rf, g[:, None], hidden)
        seg_max = m2[0]
        seg_sum = s2[0]
        best = jnp.clip(i2[0], 0, n_e - 1)

        stop_logit = (hidden @ W_stop + b_stop)[:, 0]
        st = jnp.where(done, _NEG, stop_logit)
        mm = jnp.maximum(seg_max, st)
        mm = jnp.where(jnp.isfinite(mm), mm, 0.0)
        sum_edge = seg_sum * jnp.exp(seg_max - mm)
        log_z = mm + jnp.log(sum_edge + jnp.exp(st - mm) + 1e-30)
        has_edge = seg_max > (_NEG / 2)
        choose_stop = (st >= jnp.where(has_edge, seg_max, _NEG)) | (~has_edge)
        actions = jnp.where(choose_stop, -1, best)
        log_pf = jnp.where(choose_stop, st - log_z, seg_max - log_z)
        actions = jnp.where(done, -1, actions)
        log_pf = jnp.where(done, 0.0, log_pf)
        log_pf_steps.append(log_pf)
        actions_steps.append(actions)

        sel_edge = jnp.clip(actions, 0, n_e - 1)
        sel_tail = jnp.take(node_tokens, jnp.take(tails, sel_edge), axis=0)
        sel_tail = jnp.where((actions >= 0)[:, None], sel_tail, 0.0)
        new_hidden = jnp.tanh(jnp.concatenate([hidden, sel_tail], axis=-1) @ W_upd + b_upd)
        step_mask = ((actions >= 0) & (~done))[:, None]
        hidden = jnp.where(step_mask, new_hidden, hidden)
        done = done | choose_stop

    log_pf_steps_t = jnp.stack(log_pf_steps, axis=1)
    actions_seq = jnp.stack(actions_steps, axis=1)
    log_pf_total = jnp.sum(log_pf_steps_t, axis=1)
    length = jnp.sum((actions_seq >= 0).astype(jnp.float32), axis=1)
    return log_pf_total, log_pf_steps_t, actions_seq, length


# speedup vs baseline: 17.1544x; 17.1544x over previous
import functools
import math

import jax
import jax.numpy as jnp
from jax.experimental import pallas as pl

_NEG = -1e9
_STEPS = 4
_TILE = 12800


def _rf_kernel(x_ref, w_ref, o_ref):
    o_ref[...] = jnp.dot(x_ref[...], w_ref[...], preferred_element_type=jnp.float32)


def _u_kernel(node_ref, wt_ref, wh_ref, h_ref, o_ref):
    dn = (((1,), (1,)), ((), ()))
    v_t = jax.lax.dot_general(wt_ref[...], h_ref[...], dn, preferred_element_type=jnp.float32)
    v_h = jax.lax.dot_general(wh_ref[...], h_ref[...], dn, preferred_element_type=jnp.float32)
    v = jnp.concatenate([v_t, v_h], axis=1)
    o_ref[...] = jnp.dot(node_ref[...], v, preferred_element_type=jnp.float32)


def _step_kernel(ptr_ref, rf_ref, g_ref, h_ref, m_ref, s_ref, i_ref, *, tile, n_b, n_e, inv_sqrt_d):
    t = pl.program_id(0)

    @pl.when(t == 0)
    def _():
        m_ref[...] = jnp.full_like(m_ref[...], _NEG)
        s_ref[...] = jnp.zeros_like(s_ref[...])
        i_ref[...] = jnp.full_like(i_ref[...], n_e)

    rows = t * tile + jax.lax.broadcasted_iota(jnp.int32, (tile, 1), 0)
    starts = ptr_ref[0:1, 0:n_b]
    ends = ptr_ref[0:1, 1:n_b + 1]
    mask = (rows >= starts) & (rows < ends)

    scores = (jnp.dot(rf_ref[...], h_ref[...].T, preferred_element_type=jnp.float32)
              + g_ref[...]) * inv_sqrt_d
    s = jnp.where(mask, scores, _NEG)
    tile_m = jnp.max(s, axis=0, keepdims=True)
    tile_s = jnp.sum(jnp.where(mask, jnp.exp(s - tile_m), 0.0), axis=0, keepdims=True)
    cand = jnp.where((s >= tile_m) & mask, jnp.broadcast_to(rows, s.shape), n_e)
    tile_i = jnp.min(cand, axis=0, keepdims=True)

    m_old = m_ref[...]
    s_old = s_ref[...]
    i_old = i_ref[...]
    m_new = jnp.maximum(m_old, tile_m)
    s_ref[...] = s_old * jnp.exp(m_old - m_new) + tile_s * jnp.exp(tile_m - m_new)
    i_ref[...] = jnp.where(tile_m > m_old, tile_i, i_old)
    m_ref[...] = m_new


def kernel(node_tokens, relation_tokens, question_tokens, edge_index, edge_batch, edge_ptr, node_ptr, curr_nodes, W_init, b_init, W_rel, W_tail, W_head, W_stop, b_stop, W_upd, b_upd):
    del node_ptr
    n_nodes, d = node_tokens.shape
    n_e = relation_tokens.shape[0]
    n_b = question_tokens.shape[0]
    inv_sqrt_d = 1.0 / math.sqrt(float(d))
    tile = _TILE
    n_tiles = n_e // tile

    heads = edge_index[0].astype(jnp.int32)
    tails = edge_index[1].astype(jnp.int32)
    edge_batch = edge_batch.astype(jnp.int32)

    rf = pl.pallas_call(
        _rf_kernel,
        grid=(n_tiles,),
        in_specs=[
            pl.BlockSpec((tile, d), lambda i: (i, 0)),
            pl.BlockSpec((d, d), lambda i: (0, 0)),
        ],
        out_specs=pl.BlockSpec((tile, d), lambda i: (i, 0)),
        out_shape=jax.ShapeDtypeStruct((n_e, d), jnp.float32),
    )(relation_tokens, W_rel)

    ptr_pad = jnp.zeros((1, 32), jnp.int32).at[0, :n_b + 1].set(edge_ptr.astype(jnp.int32))

    start_tok = jnp.take(node_tokens, jnp.clip(curr_nodes, 0, None), axis=0)
    start_tok = jnp.where((curr_nodes >= 0)[:, None], start_tok, 0.0)
    hidden = jnp.tanh(jnp.concatenate([question_tokens, start_tok], axis=-1) @ W_init + b_init)
    done = jnp.zeros((n_b,), dtype=bool)

    u_call = pl.pallas_call(
        _u_kernel,
        in_specs=[pl.BlockSpec(node_tokens.shape, lambda: (0, 0)),
                  pl.BlockSpec((d, d), lambda: (0, 0)),
                  pl.BlockSpec((d, d), lambda: (0, 0)),
                  pl.BlockSpec((n_b, d), lambda: (0, 0))],
        out_specs=pl.BlockSpec((n_nodes, 2 * n_b), lambda: (0, 0)),
        out_shape=jax.ShapeDtypeStruct((n_nodes, 2 * n_b), jnp.float32),
    )

    step_call = pl.pallas_call(
        functools.partial(_step_kernel, tile=tile, n_b=n_b, n_e=n_e, inv_sqrt_d=inv_sqrt_d),
        grid=(n_tiles,),
        in_specs=[
            pl.BlockSpec((1, 32), lambda i: (0, 0)),
            pl.BlockSpec((tile, d), lambda i: (i, 0)),
            pl.BlockSpec((tile, 1), lambda i: (i, 0)),
            pl.BlockSpec((n_b, d), lambda i: (0, 0)),
        ],
        out_specs=[
            pl.BlockSpec((1, n_b), lambda i: (0, 0)),
            pl.BlockSpec((1, n_b), lambda i: (0, 0)),
            pl.BlockSpec((1, n_b), lambda i: (0, 0)),
        ],
        out_shape=[
            jax.ShapeDtypeStruct((1, n_b), jnp.float32),
            jax.ShapeDtypeStruct((1, n_b), jnp.float32),
            jax.ShapeDtypeStruct((1, n_b), jnp.int32),
        ],
    )

    flat_tail = tails * (2 * n_b) + edge_batch
    flat_head = heads * (2 * n_b) + n_b + edge_batch

    log_pf_steps = []
    actions_steps = []
    for _ in range(_STEPS):
        u = u_call(node_tokens, W_tail, W_head, hidden)
        g = (jnp.take(u.reshape(-1), flat_tail) + jnp.take(u.reshape(-1), flat_head))
        m2, s2, i2 = step_call(ptr_pad, rf, g[:, None], hidden)
        seg_max = m2[0]
        seg_sum = s2[0]
        best = jnp.clip(i2[0], 0, n_e - 1)

        stop_logit = (hidden @ W_stop + b_stop)[:, 0]
        st = jnp.where(done, _NEG, stop_logit)
        mm = jnp.maximum(seg_max, st)
        mm = jnp.where(jnp.isfinite(mm), mm, 0.0)
        sum_edge = seg_sum * jnp.exp(seg_max - mm)
        log_z = mm + jnp.log(sum_edge + jnp.exp(st - mm) + 1e-30)
        has_edge = seg_max > (_NEG / 2)
        choose_stop = (st >= jnp.where(has_edge, seg_max, _NEG)) | (~has_edge)
        actions = jnp.where(choose_stop, -1, best)
        log_pf = jnp.where(choose_stop, st - log_z, seg_max - log_z)
        actions = jnp.where(done, -1, actions)
        log_pf = jnp.where(done, 0.0, log_pf)
        log_pf_steps.append(log_pf)
        actions_steps.append(actions)

        sel_edge = jnp.clip(actions, 0, n_e - 1)
        sel_tail = jnp.take(node_tokens, jnp.take(tails, sel_edge), axis=0)
        sel_tail = jnp.where((actions >= 0)[:, None], sel_tail, 0.0)
        new_hidden = jnp.tanh(jnp.concatenate([hidden, sel_tail], axis=-1) @ W_upd + b_upd)
        step_mask = ((actions >= 0) & (~done))[:, None]
        hidden = jnp.where(step_mask, new_hidden, hidden)
        done = done | choose_stop

    log_pf_steps_t = jnp.stack(log_pf_steps, axis=1)
    actions_seq = jnp.stack(actions_steps, axis=1)
    log_pf_total = jnp.sum(log_pf_steps_t, axis=1)
    length = jnp.sum((actions_seq >= 0).astype(jnp.float32), axis=1)
    return log_pf_total, log_pf_steps_t, actions_seq, length
